# baseline (speedup 1.0000x reference)
import functools

import jax
import jax.numpy as jnp
import numpy as np
from jax import lax
from jax.experimental import pallas as pl
from jax.experimental.pallas import tpu as pltpu
from jax.experimental.pallas import tpu_sc as plsc

B, S, D, H, E, HID, TOPK = 1, 2048, 768, 12, 64, 1536, 2
DH = D // H
T = B * S
NQ = 8
BQ = S // NQ
BLK = 64
NB = T * TOPK // BLK + E
PROWS = NB * BLK
NW = 32


def _rope_full_tables():
    inv = 1.0 / (10000.0 ** (np.arange(0, DH, 2, dtype=np.float32) / DH))
    t = np.arange(S, dtype=np.float32)
    fr = np.outer(t, inv)
    cos = np.concatenate([np.cos(fr), np.cos(fr)], axis=-1)
    sin = np.concatenate([np.sin(fr), np.sin(fr)], axis=-1)
    cosf = np.tile(cos, (1, H))
    sinf = np.tile(sin, (1, H))
    return cosf, sinf


_COSF, _SINF = _rope_full_tables()


def _rot_cols(w):
    w4 = w.reshape(D, H, 2, DH // 2)
    return jnp.concatenate([-w4[:, :, 1], w4[:, :, 0]], axis=2).reshape(D, D)


def _qkv_body(x_ref, w1_ref, wq_ref, wqr_ref, wk_ref, wkr_ref, wv_ref,
              cos_ref, sin_ref, q_ref, k_ref, v_ref):
    xb = x_ref[...]
    ms = jnp.mean(xb * xb, axis=1, keepdims=True)
    h = xb * lax.rsqrt(ms + 1e-6) * w1_ref[...]
    cosb = cos_ref[...]
    sinb = sin_ref[...]
    q = jnp.dot(h, wq_ref[...], preferred_element_type=jnp.float32)
    qr = jnp.dot(h, wqr_ref[...], preferred_element_type=jnp.float32)
    q_ref[...] = q * cosb + qr * sinb
    k = jnp.dot(h, wk_ref[...], preferred_element_type=jnp.float32)
    kr = jnp.dot(h, wkr_ref[...], preferred_element_type=jnp.float32)
    k_ref[...] = k * cosb + kr * sinb
    v_ref[...] = jnp.dot(h, wv_ref[...], preferred_element_type=jnp.float32)


def _qkv(x, rms1_w, wq, wqr, wk, wkr, wv):
    row = pl.BlockSpec((BQ, D), lambda i: (i, 0))
    full = pl.BlockSpec((D, D), lambda i: (0, 0))
    return pl.pallas_call(
        _qkv_body,
        grid=(NQ,),
        in_specs=[row, pl.BlockSpec((1, D), lambda i: (0, 0)),
                  full, full, full, full, full, row, row],
        out_specs=[row, row, row],
        out_shape=[jax.ShapeDtypeStruct((S, D), jnp.float32)] * 3,
    )(x, rms1_w.reshape(1, D), wq, wqr, wk, wkr, wv, _COSF, _SINF)


def _attn_body(q_ref, k_ref, v_ref, o_ref):
    qi = pl.program_id(1)
    qb = q_ref[0]
    scale = 1.0 / np.sqrt(DH).astype(np.float32)
    rows = qi * BQ + lax.broadcasted_iota(jnp.int32, (BQ, BQ), 0)
    iota_c = lax.broadcasted_iota(jnp.int32, (BQ, BQ), 1)

    def body(j, carry):
        acc, m, l = carry
        kb = k_ref[0, pl.ds(j * BQ, BQ), :]
        s = lax.dot_general(qb, kb, (((1,), (1,)), ((), ())),
                            preferred_element_type=jnp.float32) * scale
        s = jnp.where(j * BQ + iota_c <= rows, s, -1e30)
        m_new = jnp.maximum(m, jnp.max(s, axis=1, keepdims=True))
        p = jnp.exp(s - m_new)
        alpha = jnp.exp(m - m_new)
        l = l * alpha + jnp.sum(p, axis=1, keepdims=True)
        vb = v_ref[0, pl.ds(j * BQ, BQ), :]
        acc = acc * alpha + jnp.dot(p, vb, preferred_element_type=jnp.float32)
        return acc, m_new, l

    acc0 = jnp.zeros((BQ, DH), jnp.float32)
    m0 = jnp.full((BQ, 1), -3.0e38, jnp.float32)
    l0 = jnp.zeros((BQ, 1), jnp.float32)
    acc, m, l = lax.fori_loop(0, qi + 1, body, (acc0, m0, l0))
    o_ref[0] = acc / l


def _attn(q, k, v):
    qspec = pl.BlockSpec((1, BQ, DH), lambda h, qi: (h, qi, 0))
    kspec = pl.BlockSpec((1, S, DH), lambda h, qi: (h, 0, 0))
    return pl.pallas_call(
        _attn_body,
        grid=(H, NQ),
        in_specs=[qspec, kspec, kspec],
        out_specs=qspec,
        out_shape=jax.ShapeDtypeStruct((H, S, DH), jnp.float32),
    )(q, k, v)


def _gate_body(x_ref, o_ref, wo_ref, w2_ref, wg_ref, tri_ref,
               h1_ref, z_ref, ti0_ref, ti1_ref, gv0_ref, gv1_ref,
               r0_ref, r1_ref, cnt_ref, gsum_ref, psum_ref, aux_ref):
    i = pl.program_id(0)
    h1 = x_ref[...] + jnp.dot(o_ref[...], wo_ref[...],
                              preferred_element_type=jnp.float32)
    h1_ref[...] = h1
    ms = jnp.mean(h1 * h1, axis=1, keepdims=True)
    z = h1 * lax.rsqrt(ms + 1e-6) * w2_ref[...]
    z_ref[...] = z
    lg = jnp.dot(z, wg_ref[...], preferred_element_type=jnp.float32)
    lane = lax.broadcasted_iota(jnp.int32, (BQ, E), 1)
    big = jnp.int32(2 ** 30)
    m1 = jnp.max(lg, axis=1, keepdims=True)
    i1 = jnp.min(jnp.where(lg == m1, lane, big), axis=1, keepdims=True)
    lg2 = jnp.where(lane == i1, -3.0e38, lg)
    m2 = jnp.max(lg2, axis=1, keepdims=True)
    i2 = jnp.min(jnp.where(lg2 == m2, lane, big), axis=1, keepdims=True)
    ex = jnp.exp(m2 - m1)
    g1 = ex / (1.0 + ex)
    g0 = 1.0 - g1
    ti0_ref[...] = i1
    ti1_ref[...] = i2
    gv0_ref[...] = g0
    gv1_ref[...] = g1
    pe = jnp.exp(lg - m1)
    probs = pe / jnp.sum(pe, axis=1, keepdims=True)
    ohA = (lane == i1).astype(jnp.float32)
    ohB = (lane == i2).astype(jnp.float32)
    goh = ohA * g0 + ohB * g1

    @pl.when(i == 0)
    def _():
        cnt_ref[...] = jnp.zeros_like(cnt_ref)
        gsum_ref[...] = jnp.zeros_like(gsum_ref)
        psum_ref[...] = jnp.zeros_like(psum_ref)

    ohAB = ohA + ohB
    cum_excl = jnp.dot(tri_ref[...], ohAB, preferred_element_type=jnp.float32)
    base = cnt_ref[...] + cum_excl
    r0_ref[...] = jnp.sum(ohA * base, axis=1, keepdims=True).astype(jnp.int32)
    r1_ref[...] = jnp.sum(ohB * base, axis=1, keepdims=True).astype(jnp.int32)
    cnt_ref[...] += jnp.sum(ohAB, axis=0, keepdims=True)

    gsum_ref[...] += jnp.sum(goh, axis=0, keepdims=True)
    psum_ref[...] += jnp.sum(probs, axis=0, keepdims=True)

    @pl.when(i == NQ - 1)
    def _():
        aux_ref[...] = jnp.sum(gsum_ref[...] * psum_ref[...], axis=(0, 1),
                               keepdims=True) * (E / (T * T))


_TRI = np.tril(np.ones((BQ, BQ), np.float32), -1)


def _gate(x, o, wo, rms2_w, w_gate):
    row = pl.BlockSpec((BQ, D), lambda i: (i, 0))
    col = pl.BlockSpec((BQ, 1), lambda i: (i, 0))
    acc = pl.BlockSpec((1, E), lambda i: (0, 0))
    return pl.pallas_call(
        _gate_body,
        grid=(NQ,),
        in_specs=[row, row, pl.BlockSpec((D, D), lambda i: (0, 0)),
                  pl.BlockSpec((1, D), lambda i: (0, 0)),
                  pl.BlockSpec((D, E), lambda i: (0, 0)),
                  pl.BlockSpec((BQ, BQ), lambda i: (0, 0))],
        out_specs=[row, row, col, col, col, col, col, col, acc, acc, acc,
                   pl.BlockSpec((1, 1), lambda i: (0, 0))],
        out_shape=[jax.ShapeDtypeStruct((S, D), jnp.float32),
                   jax.ShapeDtypeStruct((S, D), jnp.float32),
                   jax.ShapeDtypeStruct((S, 1), jnp.int32),
                   jax.ShapeDtypeStruct((S, 1), jnp.int32),
                   jax.ShapeDtypeStruct((S, 1), jnp.float32),
                   jax.ShapeDtypeStruct((S, 1), jnp.float32),
                   jax.ShapeDtypeStruct((S, 1), jnp.int32),
                   jax.ShapeDtypeStruct((S, 1), jnp.int32),
                   jax.ShapeDtypeStruct((1, E), jnp.float32),
                   jax.ShapeDtypeStruct((1, E), jnp.float32),
                   jax.ShapeDtypeStruct((1, E), jnp.float32),
                   jax.ShapeDtypeStruct((1, 1), jnp.float32)],
    )(x, o, wo, rms2_w.reshape(1, D), w_gate, _TRI)


def _dispatch_gather(z, gidx):
    rpw = PROWS // NW
    chunk = 128

    @functools.partial(
        pl.kernel,
        mesh=plsc.VectorSubcoreMesh(core_axis_name="c", subcore_axis_name="s"),
        out_type=jax.ShapeDtypeStruct((PROWS, D), jnp.float32),
        scratch_types=[pltpu.VMEM((chunk,), jnp.int32),
                       pltpu.VMEM((chunk, D), jnp.float32),
                       pltpu.SemaphoreType.DMA],
    )
    def k(z_hbm, idx_hbm, out_hbm, idx_v, rows_v, sem):
        wid = lax.axis_index("s") * 2 + lax.axis_index("c")
        for c in range(rpw // chunk):
            base = wid * rpw + c * chunk
            pltpu.sync_copy(idx_hbm.at[pl.ds(base, chunk)], idx_v)
            pltpu.async_copy(z_hbm.at[idx_v], rows_v, sem).wait()
            pltpu.sync_copy(rows_v, out_hbm.at[pl.ds(base, chunk)])

    return k(z, gidx)


def _combine_gather(y_pad, geven, godd):
    rpw = T // NW

    @functools.partial(
        pl.kernel,
        mesh=plsc.VectorSubcoreMesh(core_axis_name="c", subcore_axis_name="s"),
        out_type=[jax.ShapeDtypeStruct((T, D), jnp.float32),
                  jax.ShapeDtypeStruct((T, D), jnp.float32)],
        scratch_types=[pltpu.VMEM((rpw,), jnp.int32),
                       pltpu.VMEM((rpw, D), jnp.float32),
                       pltpu.SemaphoreType.DMA],
    )
    def k(y_hbm, ge_hbm, go_hbm, ye_hbm, yo_hbm, idx_v, rows_v, sem):
        wid = lax.axis_index("s") * 2 + lax.axis_index("c")
        base = wid * rpw
        pltpu.sync_copy(ge_hbm.at[pl.ds(base, rpw)], idx_v)
        pltpu.async_copy(y_hbm.at[idx_v], rows_v, sem).wait()
        pltpu.sync_copy(rows_v, ye_hbm.at[pl.ds(base, rpw)])
        pltpu.sync_copy(go_hbm.at[pl.ds(base, rpw)], idx_v)
        pltpu.async_copy(y_hbm.at[idx_v], rows_v, sem).wait()
        pltpu.sync_copy(rows_v, yo_hbm.at[pl.ds(base, rpw)])

    return k(y_pad, geven, godd)


def _ffn_body(be_ref, nv_ref, z_ref, wg_ref, wu_ref, wd_ref, y_ref):
    b = pl.program_id(0)

    @pl.when(nv_ref[b] > 0)
    def _():
        zb = z_ref[...]
        g = jnp.dot(zb, wg_ref[0], preferred_element_type=jnp.float32)
        u = jnp.dot(zb, wu_ref[0], preferred_element_type=jnp.float32)
        hb = g * (1.0 / (1.0 + jnp.exp(-g))) * u
        y_ref[...] = jnp.dot(hb, wd_ref[0], preferred_element_type=jnp.float32)


def _ffn(z_pad, Wg, Wu, Wd, be, nv):
    grid_spec = pltpu.PrefetchScalarGridSpec(
        num_scalar_prefetch=2,
        grid=(NB,),
        in_specs=[
            pl.BlockSpec((BLK, D), lambda b, be, nv: (b, 0)),
            pl.BlockSpec((1, D, HID), lambda b, be, nv: (be[b], 0, 0)),
            pl.BlockSpec((1, D, HID), lambda b, be, nv: (be[b], 0, 0)),
            pl.BlockSpec((1, HID, D), lambda b, be, nv: (be[b], 0, 0)),
        ],
        out_specs=pl.BlockSpec((BLK, D), lambda b, be, nv: (b, 0)),
    )
    return pl.pallas_call(
        _ffn_body,
        grid_spec=grid_spec,
        out_shape=jax.ShapeDtypeStruct((PROWS, D), jnp.float32),
    )(be, nv, z_pad, Wg, Wu, Wd)


def _combine_body(h1_ref, ye_ref, yo_ref, g0_ref, g1_ref, out_ref):
    out_ref[...] = (h1_ref[...] + g0_ref[...] * ye_ref[...]
                    + g1_ref[...] * yo_ref[...])


def _combine(h1, y_even, y_odd, gv0, gv1):
    row = pl.BlockSpec((BQ, D), lambda i: (i, 0))
    col = pl.BlockSpec((BQ, 1), lambda i: (i, 0))
    return pl.pallas_call(
        _combine_body,
        grid=(NQ,),
        in_specs=[row, row, row, col, col],
        out_specs=row,
        out_shape=jax.ShapeDtypeStruct((S, D), jnp.float32),
    )(h1, y_even, y_odd, gv0, gv1)


def kernel(x, rms1_w, rms2_w, wq, wk, wv, wo, w_gate, Wg, Wu, Wd):
    xf = x.reshape(S, D)
    wqr = _rot_cols(wq)
    wkr = _rot_cols(wk)
    q, k, v = _qkv(xf, rms1_w, wq, wqr, wk, wkr, wv)
    qh = q.reshape(S, H, DH).transpose(1, 0, 2)
    kh = k.reshape(S, H, DH).transpose(1, 0, 2)
    vh = v.reshape(S, H, DH).transpose(1, 0, 2)
    o = _attn(qh, kh, vh).transpose(1, 0, 2).reshape(S, D)
    (h1, z, ti0, ti1, gv0, gv1, r0, r1,
     cntf, _gs, _ps, aux) = _gate(xf, o, wo, rms2_w, w_gate)

    counts = cntf.reshape(E).astype(jnp.int32)
    nblocks = (counts + BLK - 1) // BLK
    cnb = jnp.cumsum(nblocks)
    pstart = (BLK * (cnb - nblocks)).astype(jnp.int32)
    ps0 = pstart[ti0[:, 0]] + r0[:, 0]
    ps1 = pstart[ti1[:, 0]] + r1[:, 0]
    tok = jnp.arange(T, dtype=jnp.int32)
    gidx = (jnp.arange(PROWS, dtype=jnp.int32) % T).at[ps0].set(tok)
    gidx = gidx.at[ps1].set(tok)
    geven, godd = ps0, ps1
    total_blocks = cnb[-1]
    bids = jnp.arange(NB, dtype=jnp.int32)
    be_raw = jnp.searchsorted(cnb, bids, side="right").astype(jnp.int32)
    e_last = jnp.max(jnp.where(counts > 0, jnp.arange(E, dtype=jnp.int32), -1))
    be = jnp.where(bids < total_blocks, jnp.minimum(be_raw, E - 1), e_last)
    nv = (bids < total_blocks).astype(jnp.int32)

    z_pad = _dispatch_gather(z, gidx)
    y_pad = _ffn(z_pad, Wg, Wu, Wd, be, nv)
    y_even, y_odd = _combine_gather(y_pad, geven, godd)
    out = _combine(h1, y_even, y_odd, gv0, gv1)
    return out.reshape(B, S, D), aux[0, 0]


# speedup vs baseline: 1.1422x; 1.1422x over previous
import functools

import jax
import jax.numpy as jnp
import numpy as np
from jax import lax
from jax.experimental import pallas as pl
from jax.experimental.pallas import tpu as pltpu
from jax.experimental.pallas import tpu_sc as plsc

B, S, D, H, E, HID, TOPK = 1, 2048, 768, 12, 64, 1536, 2
DH = D // H
T = B * S
NQ = 8
BQ = S // NQ
BLK = 64
NB = T * TOPK // BLK + E
PROWS = NB * BLK
NW = 32


def _rope_full_tables():
    inv = 1.0 / (10000.0 ** (np.arange(0, DH, 2, dtype=np.float32) / DH))
    t = np.arange(S, dtype=np.float32)
    fr = np.outer(t, inv)
    cos = np.concatenate([np.cos(fr), np.cos(fr)], axis=-1)
    sin = np.concatenate([np.sin(fr), np.sin(fr)], axis=-1)
    cosf = np.tile(cos, (1, H))
    sinf = np.tile(sin, (1, H))
    return cosf, sinf


_COSF, _SINF = _rope_full_tables()


def _rot_cols(w):
    w4 = w.reshape(D, H, 2, DH // 2)
    return jnp.concatenate([-w4[:, :, 1], w4[:, :, 0]], axis=2).reshape(D, D)


def _qkv_body(x_ref, w1_ref, wq_ref, wqr_ref, wk_ref, wkr_ref, wv_ref,
              cos_ref, sin_ref, q_ref, k_ref, v_ref):
    xb = x_ref[...]
    ms = jnp.mean(xb * xb, axis=1, keepdims=True)
    h = xb * lax.rsqrt(ms + 1e-6) * w1_ref[...]
    cosb = cos_ref[...]
    sinb = sin_ref[...]
    q = jnp.dot(h, wq_ref[...], preferred_element_type=jnp.float32)
    qr = jnp.dot(h, wqr_ref[...], preferred_element_type=jnp.float32)
    q_ref[...] = (q * cosb + qr * sinb).astype(jnp.bfloat16)
    k = jnp.dot(h, wk_ref[...], preferred_element_type=jnp.float32)
    kr = jnp.dot(h, wkr_ref[...], preferred_element_type=jnp.float32)
    k_ref[...] = (k * cosb + kr * sinb).astype(jnp.bfloat16)
    v_ref[...] = jnp.dot(h, wv_ref[...],
                         preferred_element_type=jnp.float32).astype(jnp.bfloat16)


def _qkv(x, rms1_w, wq, wqr, wk, wkr, wv):
    row = pl.BlockSpec((BQ, D), lambda i: (i, 0))
    full = pl.BlockSpec((D, D), lambda i: (0, 0))
    return pl.pallas_call(
        _qkv_body,
        grid=(NQ,),
        in_specs=[row, pl.BlockSpec((1, D), lambda i: (0, 0)),
                  full, full, full, full, full, row, row],
        out_specs=[row, row, row],
        out_shape=[jax.ShapeDtypeStruct((S, D), jnp.bfloat16)] * 3,
    )(x, rms1_w.reshape(1, D), wq, wqr, wk, wkr, wv, _COSF, _SINF)


def _attn_body(q_ref, k_ref, v_ref, o_ref):
    qi = pl.program_id(1)
    s = lax.dot_general(q_ref[0], k_ref[0],
                        (((1,), (1,)), ((), ())),
                        preferred_element_type=jnp.float32)
    s = s * (1.0 / np.sqrt(DH).astype(np.float32))
    rows = qi * BQ + lax.broadcasted_iota(jnp.int32, (BQ, S), 0)
    cols = lax.broadcasted_iota(jnp.int32, (BQ, S), 1)
    s = jnp.where(cols <= rows, s, -1e30)
    m = jnp.max(s, axis=1, keepdims=True)
    p = jnp.exp(s - m)
    l = jnp.sum(p, axis=1, keepdims=True)
    o = jnp.dot(p.astype(jnp.bfloat16), v_ref[0],
                preferred_element_type=jnp.float32)
    o_ref[0] = o / l


def _attn(q, k, v):
    qspec = pl.BlockSpec((1, BQ, DH), lambda h, qi: (h, qi, 0))
    kspec = pl.BlockSpec((1, S, DH), lambda h, qi: (h, 0, 0))
    return pl.pallas_call(
        _attn_body,
        grid=(H, NQ),
        in_specs=[qspec, kspec, kspec],
        out_specs=pl.BlockSpec((1, BQ, DH), lambda h, qi: (h, qi, 0)),
        out_shape=jax.ShapeDtypeStruct((H, S, DH), jnp.float32),
    )(q, k, v)


def _gate_body(x_ref, o_ref, wo_ref, w2_ref, wg_ref, tri_ref,
               h1_ref, z_ref, ti0_ref, ti1_ref, gv0_ref, gv1_ref,
               r0_ref, r1_ref, cnt_ref, gsum_ref, psum_ref, aux_ref):
    i = pl.program_id(0)
    h1 = x_ref[...] + jnp.dot(o_ref[...], wo_ref[...],
                              preferred_element_type=jnp.float32)
    h1_ref[...] = h1
    ms = jnp.mean(h1 * h1, axis=1, keepdims=True)
    z = h1 * lax.rsqrt(ms + 1e-6) * w2_ref[...]
    z_ref[...] = z
    lg = jnp.dot(z, wg_ref[...], preferred_element_type=jnp.float32)
    lane = lax.broadcasted_iota(jnp.int32, (BQ, E), 1)
    big = jnp.int32(2 ** 30)
    m1 = jnp.max(lg, axis=1, keepdims=True)
    i1 = jnp.min(jnp.where(lg == m1, lane, big), axis=1, keepdims=True)
    lg2 = jnp.where(lane == i1, -3.0e38, lg)
    m2 = jnp.max(lg2, axis=1, keepdims=True)
    i2 = jnp.min(jnp.where(lg2 == m2, lane, big), axis=1, keepdims=True)
    ex = jnp.exp(m2 - m1)
    g1 = ex / (1.0 + ex)
    g0 = 1.0 - g1
    ti0_ref[...] = i1
    ti1_ref[...] = i2
    gv0_ref[...] = g0
    gv1_ref[...] = g1
    pe = jnp.exp(lg - m1)
    probs = pe / jnp.sum(pe, axis=1, keepdims=True)
    ohA = (lane == i1).astype(jnp.float32)
    ohB = (lane == i2).astype(jnp.float32)
    goh = ohA * g0 + ohB * g1

    @pl.when(i == 0)
    def _():
        cnt_ref[...] = jnp.zeros_like(cnt_ref)
        gsum_ref[...] = jnp.zeros_like(gsum_ref)
        psum_ref[...] = jnp.zeros_like(psum_ref)

    ohAB = ohA + ohB
    cum_excl = jnp.dot(tri_ref[...], ohAB, preferred_element_type=jnp.float32)
    base = cnt_ref[...] + cum_excl
    r0_ref[...] = jnp.sum(ohA * base, axis=1, keepdims=True).astype(jnp.int32)
    r1_ref[...] = jnp.sum(ohB * base, axis=1, keepdims=True).astype(jnp.int32)
    cnt_ref[...] += jnp.sum(ohAB, axis=0, keepdims=True)

    gsum_ref[...] += jnp.sum(goh, axis=0, keepdims=True)
    psum_ref[...] += jnp.sum(probs, axis=0, keepdims=True)

    @pl.when(i == NQ - 1)
    def _():
        aux_ref[...] = jnp.sum(gsum_ref[...] * psum_ref[...], axis=(0, 1),
                               keepdims=True) * (E / (T * T))


_TRI = np.tril(np.ones((BQ, BQ), np.float32), -1)


def _gate(x, o, wo, rms2_w, w_gate):
    row = pl.BlockSpec((BQ, D), lambda i: (i, 0))
    col = pl.BlockSpec((BQ, 1), lambda i: (i, 0))
    acc = pl.BlockSpec((1, E), lambda i: (0, 0))
    return pl.pallas_call(
        _gate_body,
        grid=(NQ,),
        in_specs=[row, row, pl.BlockSpec((D, D), lambda i: (0, 0)),
                  pl.BlockSpec((1, D), lambda i: (0, 0)),
                  pl.BlockSpec((D, E), lambda i: (0, 0)),
                  pl.BlockSpec((BQ, BQ), lambda i: (0, 0))],
        out_specs=[row, row, col, col, col, col, col, col, acc, acc, acc,
                   pl.BlockSpec((1, 1), lambda i: (0, 0))],
        out_shape=[jax.ShapeDtypeStruct((S, D), jnp.float32),
                   jax.ShapeDtypeStruct((S, D), jnp.float32),
                   jax.ShapeDtypeStruct((S, 1), jnp.int32),
                   jax.ShapeDtypeStruct((S, 1), jnp.int32),
                   jax.ShapeDtypeStruct((S, 1), jnp.float32),
                   jax.ShapeDtypeStruct((S, 1), jnp.float32),
                   jax.ShapeDtypeStruct((S, 1), jnp.int32),
                   jax.ShapeDtypeStruct((S, 1), jnp.int32),
                   jax.ShapeDtypeStruct((1, E), jnp.float32),
                   jax.ShapeDtypeStruct((1, E), jnp.float32),
                   jax.ShapeDtypeStruct((1, E), jnp.float32),
                   jax.ShapeDtypeStruct((1, 1), jnp.float32)],
    )(x, o, wo, rms2_w.reshape(1, D), w_gate, _TRI)


def _dispatch_gather(z, gidx):
    rpw = PROWS // NW
    chunk = 128

    @functools.partial(
        pl.kernel,
        mesh=plsc.VectorSubcoreMesh(core_axis_name="c", subcore_axis_name="s"),
        out_type=jax.ShapeDtypeStruct((PROWS, D), jnp.float32),
        scratch_types=[pltpu.VMEM((chunk,), jnp.int32),
                       pltpu.VMEM((chunk, D), jnp.float32),
                       pltpu.SemaphoreType.DMA],
    )
    def k(z_hbm, idx_hbm, out_hbm, idx_v, rows_v, sem):
        wid = lax.axis_index("s") * 2 + lax.axis_index("c")
        for c in range(rpw // chunk):
            base = wid * rpw + c * chunk
            pltpu.sync_copy(idx_hbm.at[pl.ds(base, chunk)], idx_v)
            pltpu.async_copy(z_hbm.at[idx_v], rows_v, sem).wait()
            pltpu.sync_copy(rows_v, out_hbm.at[pl.ds(base, chunk)])

    return k(z, gidx)


def _combine_gather(y_pad, geven, godd):
    rpw = T // NW

    @functools.partial(
        pl.kernel,
        mesh=plsc.VectorSubcoreMesh(core_axis_name="c", subcore_axis_name="s"),
        out_type=[jax.ShapeDtypeStruct((T, D), jnp.float32),
                  jax.ShapeDtypeStruct((T, D), jnp.float32)],
        scratch_types=[pltpu.VMEM((rpw,), jnp.int32),
                       pltpu.VMEM((rpw, D), jnp.float32),
                       pltpu.SemaphoreType.DMA],
    )
    def k(y_hbm, ge_hbm, go_hbm, ye_hbm, yo_hbm, idx_v, rows_v, sem):
        wid = lax.axis_index("s") * 2 + lax.axis_index("c")
        base = wid * rpw
        pltpu.sync_copy(ge_hbm.at[pl.ds(base, rpw)], idx_v)
        pltpu.async_copy(y_hbm.at[idx_v], rows_v, sem).wait()
        pltpu.sync_copy(rows_v, ye_hbm.at[pl.ds(base, rpw)])
        pltpu.sync_copy(go_hbm.at[pl.ds(base, rpw)], idx_v)
        pltpu.async_copy(y_hbm.at[idx_v], rows_v, sem).wait()
        pltpu.sync_copy(rows_v, yo_hbm.at[pl.ds(base, rpw)])

    return k(y_pad, geven, godd)


def _ffn_body(be_ref, nv_ref, z_ref, wg_ref, wu_ref, wd_ref, y_ref):
    b = pl.program_id(0)

    @pl.when(nv_ref[b] > 0)
    def _():
        zb = z_ref[...].astype(jnp.bfloat16)
        g = jnp.dot(zb, wg_ref[0].astype(jnp.bfloat16),
                    preferred_element_type=jnp.float32)
        u = jnp.dot(zb, wu_ref[0].astype(jnp.bfloat16),
                    preferred_element_type=jnp.float32)
        hb = g * (1.0 / (1.0 + jnp.exp(-g))) * u
        y_ref[...] = jnp.dot(hb.astype(jnp.bfloat16),
                             wd_ref[0].astype(jnp.bfloat16),
                             preferred_element_type=jnp.float32)


def _ffn(z_pad, Wg, Wu, Wd, be, nv):
    grid_spec = pltpu.PrefetchScalarGridSpec(
        num_scalar_prefetch=2,
        grid=(NB,),
        in_specs=[
            pl.BlockSpec((BLK, D), lambda b, be, nv: (b, 0)),
            pl.BlockSpec((1, D, HID), lambda b, be, nv: (be[b], 0, 0)),
            pl.BlockSpec((1, D, HID), lambda b, be, nv: (be[b], 0, 0)),
            pl.BlockSpec((1, HID, D), lambda b, be, nv: (be[b], 0, 0)),
        ],
        out_specs=pl.BlockSpec((BLK, D), lambda b, be, nv: (b, 0)),
    )
    return pl.pallas_call(
        _ffn_body,
        grid_spec=grid_spec,
        out_shape=jax.ShapeDtypeStruct((PROWS, D), jnp.float32),
    )(be, nv, z_pad, Wg, Wu, Wd)


def _combine_body(h1_ref, ye_ref, yo_ref, g0_ref, g1_ref, out_ref):
    out_ref[...] = (h1_ref[...] + g0_ref[...] * ye_ref[...]
                    + g1_ref[...] * yo_ref[...])


def _combine(h1, y_even, y_odd, gv0, gv1):
    row = pl.BlockSpec((BQ, D), lambda i: (i, 0))
    col = pl.BlockSpec((BQ, 1), lambda i: (i, 0))
    return pl.pallas_call(
        _combine_body,
        grid=(NQ,),
        in_specs=[row, row, row, col, col],
        out_specs=row,
        out_shape=jax.ShapeDtypeStruct((S, D), jnp.float32),
    )(h1, y_even, y_odd, gv0, gv1)


def kernel(x, rms1_w, rms2_w, wq, wk, wv, wo, w_gate, Wg, Wu, Wd):
    xf = x.reshape(S, D)
    wqr = _rot_cols(wq)
    wkr = _rot_cols(wk)
    q, k, v = _qkv(xf, rms1_w, wq, wqr, wk, wkr, wv)
    qh = q.reshape(S, H, DH).transpose(1, 0, 2)
    kh = k.reshape(S, H, DH).transpose(1, 0, 2)
    vh = v.reshape(S, H, DH).transpose(1, 0, 2)
    o = _attn(qh, kh, vh).transpose(1, 0, 2).reshape(S, D)
    (h1, z, ti0, ti1, gv0, gv1, r0, r1,
     cntf, _gs, _ps, aux) = _gate(xf, o, wo, rms2_w, w_gate)

    counts = cntf.reshape(E).astype(jnp.int32)
    nblocks = (counts + BLK - 1) // BLK
    cnb = jnp.cumsum(nblocks)
    pstart = (BLK * (cnb - nblocks)).astype(jnp.int32)
    ps0 = pstart[ti0[:, 0]] + r0[:, 0]
    ps1 = pstart[ti1[:, 0]] + r1[:, 0]
    tok = jnp.arange(T, dtype=jnp.int32)
    gidx = (jnp.arange(PROWS, dtype=jnp.int32) % T).at[
        jnp.concatenate([ps0, ps1])].set(jnp.concatenate([tok, tok]))
    geven, godd = ps0, ps1
    total_blocks = cnb[-1]
    bids = jnp.arange(NB, dtype=jnp.int32)
    be_raw = jnp.searchsorted(cnb, bids, side="right").astype(jnp.int32)
    e_last = jnp.max(jnp.where(counts > 0, jnp.arange(E, dtype=jnp.int32), -1))
    be = jnp.where(bids < total_blocks, jnp.minimum(be_raw, E - 1), e_last)
    nv = (bids < total_blocks).astype(jnp.int32)

    z_pad = _dispatch_gather(z, gidx)
    y_pad = _ffn(z_pad, Wg, Wu, Wd, be, nv)
    y_even, y_odd = _combine_gather(y_pad, geven, godd)
    out = _combine(h1, y_even, y_odd, gv0, gv1)
    return out.reshape(B, S, D), aux[0, 0]
